# baseline (speedup 1.0000x reference)
import functools

import jax
import jax.numpy as jnp
from jax import lax
from jax.experimental import pallas as pl
from jax.experimental.pallas import tpu as pltpu
from jax.experimental.pallas import tpu_sc as plsc

_INFO = plsc.get_sparse_core_info()
_NC = _INFO.num_cores
_NS = _INFO.num_subcores
_NW = _NC * _NS

_V = 1000
_D = 128
_B = 16384
_BPW = _B // _NW

_mesh = plsc.VectorSubcoreMesh(core_axis_name="c", subcore_axis_name="s")


@functools.partial(
    pl.kernel,
    mesh=_mesh,
    out_type=jax.ShapeDtypeStruct((_B, _D), jnp.float32),
    scratch_types=[
        pltpu.VMEM((_BPW,), jnp.int32),
        pltpu.VMEM((_BPW, _D), jnp.float32),
        pltpu.VMEM_SHARED((_V, _D), jnp.float32),
        pltpu.SemaphoreType.DMA,
    ],
)
def _gather_kernel(idx_hbm, table_hbm, out_hbm, idx_v, rows_v, table_s, sem):
    sid = lax.axis_index("s")
    wid = sid * _NC + lax.axis_index("c")
    base = wid * _BPW
    for t, (off, n) in enumerate(((0, 256), (256, 256), (512, 256), (768, 232))):
        @pl.when(sid == t * 4)
        def _(off=off, n=n):
            pltpu.sync_copy(table_hbm.at[pl.ds(off, n)], table_s.at[pl.ds(off, n)])
    pltpu.sync_copy(idx_hbm.at[pl.ds(base, _BPW)], idx_v)
    plsc.subcore_barrier()
    pltpu.async_copy(table_s.at[idx_v], rows_v, sem).wait()
    pltpu.sync_copy(rows_v, out_hbm.at[pl.ds(base, _BPW)])


def kernel(indices, embedding_weight):
    return _gather_kernel(indices.astype(jnp.int32), embedding_weight)


# speedup vs baseline: 1.0016x; 1.0016x over previous
import functools

import jax
import jax.numpy as jnp
from jax import lax
from jax.experimental import pallas as pl
from jax.experimental.pallas import tpu as pltpu
from jax.experimental.pallas import tpu_sc as plsc

_INFO = plsc.get_sparse_core_info()
_NC = _INFO.num_cores
_NS = _INFO.num_subcores
_NW = _NC * _NS

_V = 1000
_D = 128
_B = 16384
_BPW = _B // _NW

_mesh = plsc.VectorSubcoreMesh(core_axis_name="c", subcore_axis_name="s")


@functools.partial(
    pl.kernel,
    mesh=_mesh,
    out_type=jax.ShapeDtypeStruct((_B, _D), jnp.float32),
    scratch_types=[
        pltpu.VMEM((_BPW,), jnp.int32),
        pltpu.VMEM((_BPW, _D), jnp.float32),
        pltpu.VMEM_SHARED((_V, _D), jnp.float32),
        pltpu.SemaphoreType.DMA,
    ],
)
def _gather_kernel(idx_hbm, table_hbm, out_hbm, idx_v, rows_v, table_s, sem):
    sid = lax.axis_index("s")
    wid = lax.axis_index("c") * _NS + sid
    base = wid * _BPW
    @pl.when(sid == 0)
    def _():
        pltpu.sync_copy(table_hbm, table_s)
    pltpu.sync_copy(idx_hbm.at[pl.ds(base, _BPW)], idx_v)
    plsc.subcore_barrier()
    pltpu.async_copy(table_s.at[idx_v], rows_v, sem).wait()
    pltpu.sync_copy(rows_v, out_hbm.at[pl.ds(base, _BPW)])


def kernel(indices, embedding_weight):
    return _gather_kernel(indices.astype(jnp.int32), embedding_weight)


# speedup vs baseline: 1.0067x; 1.0051x over previous
import functools

import jax
import jax.numpy as jnp
from jax import lax
from jax.experimental import pallas as pl
from jax.experimental.pallas import tpu as pltpu
from jax.experimental.pallas import tpu_sc as plsc

_INFO = plsc.get_sparse_core_info()
_NC = _INFO.num_cores
_NS = _INFO.num_subcores
_NW = _NC * _NS

_V = 1000
_D = 128
_B = 16384
_BPW = _B // _NW

_mesh = plsc.VectorSubcoreMesh(core_axis_name="c", subcore_axis_name="s")


@functools.partial(
    pl.kernel,
    mesh=_mesh,
    out_type=jax.ShapeDtypeStruct((_B, _D), jnp.float32),
    scratch_types=[
        pltpu.VMEM((_BPW,), jnp.int32),
        pltpu.VMEM((_BPW, _D), jnp.float32),
        pltpu.VMEM_SHARED((_V, _D), jnp.float32),
        pltpu.SemaphoreType.DMA,
    ],
)
def _gather_kernel(idx_hbm, table_hbm, out_hbm, idx_v, rows_v, table_s, sem):
    sid = lax.axis_index("s")
    wid = sid * _NC + lax.axis_index("c")
    base = wid * _BPW
    @pl.when(sid == 0)
    def _():
        pltpu.sync_copy(table_hbm, table_s)
    pltpu.sync_copy(idx_hbm.at[pl.ds(base, _BPW)], idx_v)
    plsc.subcore_barrier()
    pltpu.async_copy(table_s.at[idx_v], rows_v, sem).wait()
    pltpu.sync_copy(rows_v, out_hbm.at[pl.ds(base, _BPW)])


def kernel(indices, embedding_weight):
    return _gather_kernel(indices.astype(jnp.int32), embedding_weight)
